# Initial kernel scaffold; baseline (speedup 1.0000x reference)
import jax
import jax.numpy as jnp
from jax.experimental import pallas as pl


def kernel(feature_indices, feature_values, weight, bias):
    raise NotImplementedError("write your pallas kernel here")


# speedup vs baseline: 4.5416x; 4.5416x over previous
import functools

import jax
import jax.numpy as jnp
from jax import lax
from jax.experimental import pallas as pl
from jax.experimental.pallas import tpu as pltpu
from jax.experimental.pallas import tpu_sc as plsc

NC = 2
NS = 16
L = 16
NW = NC * NS

B = 16384
A = 100
O = 32
AP = 112
CH = 16
NCHUNK = B // CH // NW


def _body(fi_hbm, fvt_hbm, w_hbm, bias_hbm, out_hbm,
          idx_v, val_v, rows_v, out_v, bias_v, sem):
    cid = lax.axis_index("c")
    sid = lax.axis_index("s")
    wid = sid * NC + cid

    iota = lax.iota(jnp.int32, L)
    zeros = jnp.zeros((L,), jnp.float32)

    for j in range(CH):
        for i in range(A, AP):
            for oo in range(O // L):
                rows_v[j, i, pl.ds(oo * L, L)] = zeros

    pltpu.sync_copy(bias_hbm, bias_v)

    @pl.loop(0, NCHUNK)
    def _chunk(k):
        chunk_id = wid * NCHUNK + k
        base = chunk_id * CH
        pltpu.sync_copy(fi_hbm.at[pl.ds(base, CH)], idx_v)
        pltpu.sync_copy(fvt_hbm.at[chunk_id], val_v)
        cps = [
            pltpu.async_copy(w_hbm.at[idx_v.at[j]],
                             rows_v.at[j, pl.ds(0, A)], sem)
            for j in range(CH)
        ]
        for cp in cps:
            cp.wait()

        for og in range(O // L):
            def ibody(i, accs):
                v = val_v[i, :]
                ii = jnp.broadcast_to(i, (L,))
                out = []
                for oo in range(L):
                    o = og * L + oo
                    x = plsc.load_gather(
                        rows_v,
                        [iota, ii, jnp.full((L,), o, jnp.int32)])
                    out.append(accs[oo] + x * v)
                return tuple(out)

            accs0 = tuple(bias_v[og * L + oo, :] for oo in range(L))
            accs = lax.fori_loop(0, AP, ibody, accs0)
            for oo in range(L):
                plsc.store_scatter(
                    out_v, [iota, jnp.full((L,), og * L + oo, jnp.int32)],
                    accs[oo])

        pltpu.sync_copy(out_v, out_hbm.at[pl.ds(base, CH)])


@jax.jit
def _run(fi, fvt, w, bias_b):
    mesh = plsc.VectorSubcoreMesh(core_axis_name="c", subcore_axis_name="s")
    f = pl.kernel(
        _body,
        out_type=jax.ShapeDtypeStruct((B, O), jnp.float32),
        mesh=mesh,
        compiler_params=pltpu.CompilerParams(
            needs_layout_passes=False,
            use_tc_tiling_on_sc=False,
        ),
        scratch_types=[
            pltpu.VMEM((CH, A), jnp.int32),
            pltpu.VMEM((AP, CH), jnp.float32),
            pltpu.VMEM((CH, AP, O), jnp.float32),
            pltpu.VMEM((CH, O), jnp.float32),
            pltpu.VMEM((O, L), jnp.float32),
            pltpu.SemaphoreType.DMA,
        ],
    )
    return f(fi, fvt, w, bias_b)


def kernel(feature_indices, feature_values, weight, bias):
    fv = jnp.pad(feature_values, ((0, 0), (0, AP - A)))
    fvt = fv.reshape(B // CH, CH, AP).transpose(0, 2, 1)
    bias_b = jnp.broadcast_to(bias[:, None], (O, L))
    return _run(feature_indices, fvt, weight, bias_b)
